# baseline (speedup 1.0000x reference)
import functools

import jax
import jax.numpy as jnp
import numpy as np
from jax import lax
from jax.experimental import pallas as pl
from jax.experimental.pallas import tpu as pltpu
from jax.experimental.pallas import tpu_sc as plsc

SEQ = 200
EMB = 32
BATCH = 4096
VOCAB = 1000000
NW = 32

NCOLS_FULL = VOCAB // 128
TAIL = VOCAB - NCOLS_FULL * 128
COLS_PER_W = NCOLS_FULL // NW
EXTRA = NCOLS_FULL - COLS_PER_W * NW
CBLK = 4
NSTEP = COLS_PER_W // CBLK
TPS = CBLK * 128
ORPS = TPS * EMB // 128
TAILR = TAIL * EMB // 128

BLK = BATCH // 128
UNITS = SEQ * BLK
UPW = UNITS // NW
UPC = 4
NCHUNK = UPW // UPC
CROWS = UPC * 128
SCALE = float(np.sqrt(np.float32(EMB)))

_mesh = plsc.VectorSubcoreMesh(core_axis_name="c", subcore_axis_name="s")


def _transpose(tin, tout, nrows):
    def tbody(r, carry):
        for q in range(4):
            t = r * 4 + q
            ridx = lax.iota(jnp.int32, 16)
            cidx = jnp.zeros((16,), jnp.int32) + t
            tout[r, pl.ds(q * EMB, 16)] = plsc.load_gather(tin, [ridx, cidx])
            tout[r, pl.ds(q * EMB + 16, 16)] = plsc.load_gather(
                tin, [ridx + 16, cidx])
        return carry

    lax.fori_loop(0, nrows, tbody, 0)


@functools.partial(
    pl.kernel,
    out_type=jax.ShapeDtypeStruct((VOCAB // 4, 128), jnp.float32),
    mesh=_mesh,
    compiler_params=pltpu.CompilerParams(needs_layout_passes=False),
    scratch_types=[
        pltpu.VMEM((EMB, TPS), jnp.float32),
        pltpu.VMEM((EMB, TPS), jnp.float32),
        pltpu.VMEM((ORPS, 128), jnp.float32),
        pltpu.VMEM((ORPS, 128), jnp.float32),
        pltpu.VMEM((TAILR, 128), jnp.float32),
        pltpu.VMEM((EMB, 128), jnp.float32),
        pltpu.VMEM((EMB, 128), jnp.float32),
        pltpu.SemaphoreType.DMA,
        pltpu.SemaphoreType.DMA,
    ],
)
def _sc_format(tt_hbm, tail_hbm, out_hbm, tin0, tin1, tout0, tout1, tbuf,
               ein_v, eout_v, sem_in, sem_out):
    wid = lax.axis_index("s") * 2 + lax.axis_index("c")
    tins = (tin0, tin1)
    touts = (tout0, tout1)

    def fire_in(s):
        tok0 = (wid * COLS_PER_W + s * CBLK) * 128
        return pltpu.async_copy(
            tt_hbm.at[:, pl.ds(pl.multiple_of(tok0, 128), TPS)],
            tins[s % 2], sem_in)

    def fire_out(s):
        orow = (wid * COLS_PER_W + s * CBLK) * EMB
        return pltpu.async_copy(
            touts[s % 2],
            out_hbm.at[pl.ds(pl.multiple_of(orow, 32), ORPS)],
            sem_out)

    ins = [fire_in(0)]
    outs = []
    for s in range(NSTEP):
        if s + 1 < NSTEP:
            ins.append(fire_in(s + 1))
        ins[s].wait()
        if s >= 2:
            outs[s - 2].wait()
        _transpose(tins[s % 2], touts[s % 2], ORPS)
        outs.append(fire_out(s))
    outs[-2].wait()
    outs[-1].wait()

    @pl.when(wid < EXTRA)
    def _():
        tok0 = (NCOLS_FULL - EXTRA + wid) * 128
        pltpu.sync_copy(
            tt_hbm.at[:, pl.ds(pl.multiple_of(tok0, 128), 128)], ein_v)
        _transpose(ein_v, eout_v, 32)
        pltpu.sync_copy(
            eout_v,
            out_hbm.at[pl.ds(pl.multiple_of(tok0 * EMB // 128, 32), 32)])

    @pl.when(wid == EXTRA)
    def _():
        pltpu.sync_copy(tail_hbm, tbuf)
        pltpu.sync_copy(
            tbuf,
            out_hbm.at[pl.ds(pl.multiple_of(NCOLS_FULL * 128 * EMB // 128, 16),
                             TAILR)])


@functools.partial(
    pl.kernel,
    out_type=jax.ShapeDtypeStruct((SEQ, EMB, BATCH), jnp.float32),
    mesh=_mesh,
    compiler_params=pltpu.CompilerParams(needs_layout_passes=False),
    scratch_types=[
        pltpu.VMEM((CROWS,), jnp.int32),
        pltpu.VMEM((CROWS,), jnp.int32),
        pltpu.VMEM((CROWS, 128), jnp.float32),
        pltpu.VMEM((UPC, 512), jnp.float32),
        pltpu.VMEM((UPC, EMB, 128), jnp.float32),
        pltpu.SemaphoreType.DMA,
        pltpu.SemaphoreType.DMA,
    ],
)
def _sc_gather(idx_hbm, table_hbm, pos_hbm, out_hbm,
               idx_v, gidx_v, gbuf, pose_v, obuf, sem, sem_out):
    wid = lax.axis_index("s") * 2 + lax.axis_index("c")
    u0w = wid * UPW

    def chunk_body(c, carry):
        u0 = u0w + c * UPC
        pltpu.sync_copy(idx_hbm.at[pl.ds(pl.multiple_of(u0 * 128, CROWS), CROWS)],
                        idx_v)

        def gidx_body(v, carry2):
            q = pl.ds(v * 16, 16)
            gidx_v[q] = lax.shift_right_logical(idx_v[q], 2)
            return carry2

        lax.fori_loop(0, CROWS // 16, gidx_body, 0)

        copies = []
        for j in range(UPC):
            s_j = (u0 + j) // BLK
            copies.append(pltpu.async_copy(
                pos_hbm.at[s_j], pose_v.at[j], sem))
        for j in range(UPC):
            copies.append(pltpu.async_copy(
                table_hbm.at[gidx_v.at[pl.ds(j * 128, 128)]],
                gbuf.at[pl.ds(j * 128, 128)], sem))
        for cp in copies:
            cp.wait()

        out_fires = []
        for j in range(UPC):
            for b16 in range(8):
                base = j * 128 + b16 * 16
                offv = (idx_v[pl.ds(base, 16)] & 3) * EMB
                rowv = lax.iota(jnp.int32, 16) + base

                def ebody(e, carry2, offv=offv, rowv=rowv, j=j, b16=b16):
                    colv = offv + e
                    vec = plsc.load_gather(gbuf, [rowv, colv])
                    pvec = pose_v[j, pl.ds(e * 16, 16)]
                    obuf[j, e, pl.ds(b16 * 16, 16)] = vec * SCALE + pvec
                    return carry2

                lax.fori_loop(0, EMB, ebody, 0)

        for j in range(UPC):
            s_j = (u0 + j) // BLK
            blk_j = lax.rem(u0 + j, BLK)
            out_fires.append(pltpu.async_copy(
                obuf.at[j],
                out_hbm.at[s_j, :, pl.ds(pl.multiple_of(blk_j * 128, 128), 128)],
                sem_out))
        for cp in out_fires:
            cp.wait()
        return carry

    lax.fori_loop(0, NCHUNK, chunk_body, 0)


def kernel(inputs, token_table, pos_table):
    tail4 = token_table[NCOLS_FULL * 128:].reshape(TAILR, 128)
    table4 = _sc_format(token_table.T, tail4)
    idxT = inputs.T.reshape(-1).astype(jnp.int32)
    pos_exp = jnp.broadcast_to(
        pos_table[:, :, None], (SEQ, EMB, 16)).reshape(SEQ, EMB * 16)
    out = _sc_gather(idxT, table4, pos_exp)
    return out.transpose(2, 0, 1)


# speedup vs baseline: 1.0254x; 1.0254x over previous
import functools

import jax
import jax.numpy as jnp
import numpy as np
from jax import lax
from jax.experimental import pallas as pl
from jax.experimental.pallas import tpu as pltpu
from jax.experimental.pallas import tpu_sc as plsc

SEQ = 200
EMB = 32
BATCH = 4096
VOCAB = 1000000
NW = 32

NCOLS_FULL = VOCAB // 128
TAIL = VOCAB - NCOLS_FULL * 128
COLS_PER_W = NCOLS_FULL // NW
EXTRA = NCOLS_FULL - COLS_PER_W * NW
CBLK = 4
NSTEP = COLS_PER_W // CBLK
TPS = CBLK * 128
ORPS = TPS * EMB // 128
TAILR = TAIL * EMB // 128

BLK = BATCH // 128
UNITS = SEQ * BLK
UPW = UNITS // NW
UPC = 4
NCHUNK = UPW // UPC
CROWS = UPC * 128
SCALE = float(np.sqrt(np.float32(EMB)))

_mesh = plsc.VectorSubcoreMesh(core_axis_name="c", subcore_axis_name="s")


def _transpose(tin, tout, nrows):
    def tbody(r, carry):
        for q in range(4):
            t = r * 4 + q
            ridx = lax.iota(jnp.int32, 16)
            cidx = jnp.zeros((16,), jnp.int32) + t
            tout[r, pl.ds(q * EMB, 16)] = plsc.load_gather(tin, [ridx, cidx])
            tout[r, pl.ds(q * EMB + 16, 16)] = plsc.load_gather(
                tin, [ridx + 16, cidx])
        return carry

    lax.fori_loop(0, nrows, tbody, 0, unroll=4)


@functools.partial(
    pl.kernel,
    out_type=jax.ShapeDtypeStruct((VOCAB // 4, 128), jnp.float32),
    mesh=_mesh,
    compiler_params=pltpu.CompilerParams(needs_layout_passes=False),
    scratch_types=[
        pltpu.VMEM((2, EMB, TPS), jnp.float32),
        pltpu.VMEM((2, ORPS, 128), jnp.float32),
        pltpu.VMEM((TAILR, 128), jnp.float32),
        pltpu.VMEM((EMB, 128), jnp.float32),
        pltpu.VMEM((EMB, 128), jnp.float32),
        pltpu.SemaphoreType.DMA,
        pltpu.SemaphoreType.DMA,
    ],
)
def _sc_format(tt_hbm, tail_hbm, out_hbm, tin3, tout3, tbuf,
               ein_v, eout_v, sem_in, sem_out):
    wid = lax.axis_index("s") * 2 + lax.axis_index("c")

    def fire_in(s):
        tok0 = (wid * COLS_PER_W + s * CBLK) * 128
        return pltpu.async_copy(
            tt_hbm.at[:, pl.ds(pl.multiple_of(tok0, 128), TPS)],
            tin3.at[lax.rem(s, 2)], sem_in)

    def fire_out(s):
        orow = (wid * COLS_PER_W + s * CBLK) * EMB
        return pltpu.async_copy(
            tout3.at[lax.rem(s, 2)],
            out_hbm.at[pl.ds(pl.multiple_of(orow, 32), ORPS)],
            sem_out)

    def drain_in():
        pltpu.make_async_copy(
            tt_hbm.at[:, pl.ds(0, TPS)], tin3.at[0], sem_in).wait()

    def drain_out():
        pltpu.make_async_copy(
            tout3.at[0], out_hbm.at[pl.ds(0, ORPS)], sem_out).wait()

    fire_in(0)

    def step_body(s, carry):
        p = lax.rem(s, 2)

        @pl.when(s + 1 < NSTEP)
        def _():
            fire_in(s + 1)

        drain_in()

        @pl.when(s >= 2)
        def _():
            drain_out()

        def tbody(r, carry2):
            for q in range(4):
                t = r * 4 + q
                ridx = lax.iota(jnp.int32, 16)
                cidx = jnp.zeros((16,), jnp.int32) + t
                pv = jnp.zeros((16,), jnp.int32) + p
                tout3[p, r, pl.ds(q * EMB, 16)] = plsc.load_gather(
                    tin3, [pv, ridx, cidx])
                tout3[p, r, pl.ds(q * EMB + 16, 16)] = plsc.load_gather(
                    tin3, [pv, ridx + 16, cidx])
            return carry2

        lax.fori_loop(0, ORPS, tbody, 0, unroll=4)
        fire_out(s)
        return carry

    lax.fori_loop(0, NSTEP, step_body, 0)
    drain_out()
    drain_out()

    @pl.when(wid < EXTRA)
    def _():
        tok0 = (NCOLS_FULL - EXTRA + wid) * 128
        pltpu.sync_copy(
            tt_hbm.at[:, pl.ds(pl.multiple_of(tok0, 128), 128)], ein_v)
        _transpose(ein_v, eout_v, 32)
        pltpu.sync_copy(
            eout_v,
            out_hbm.at[pl.ds(pl.multiple_of(tok0 * EMB // 128, 32), 32)])

    @pl.when(wid == EXTRA)
    def _():
        pltpu.sync_copy(tail_hbm, tbuf)
        pltpu.sync_copy(
            tbuf,
            out_hbm.at[pl.ds(pl.multiple_of(NCOLS_FULL * 128 * EMB // 128, 16),
                             TAILR)])


@functools.partial(
    pl.kernel,
    out_type=jax.ShapeDtypeStruct((SEQ, EMB, BATCH), jnp.float32),
    mesh=_mesh,
    compiler_params=pltpu.CompilerParams(needs_layout_passes=False),
    scratch_types=[
        pltpu.VMEM((CROWS,), jnp.int32),
        pltpu.VMEM((CROWS,), jnp.int32),
        pltpu.VMEM((CROWS, 128), jnp.float32),
        pltpu.VMEM((UPC, 512), jnp.float32),
        pltpu.VMEM((UPC, EMB, 128), jnp.float32),
        pltpu.SemaphoreType.DMA,
        pltpu.SemaphoreType.DMA,
    ],
)
def _sc_gather(idx_hbm, table_hbm, pos_hbm, out_hbm,
               idx_v, gidx_v, gbuf, pose_v, obuf, sem, sem_out):
    wid = lax.axis_index("s") * 2 + lax.axis_index("c")
    u0w = wid * UPW

    def chunk_body(c, carry):
        u0 = u0w + c * UPC
        pltpu.sync_copy(idx_hbm.at[pl.ds(pl.multiple_of(u0 * 128, CROWS), CROWS)],
                        idx_v)

        def gidx_body(v, carry2):
            q = pl.ds(v * 16, 16)
            gidx_v[q] = lax.shift_right_logical(idx_v[q], 2)
            return carry2

        lax.fori_loop(0, CROWS // 16, gidx_body, 0, unroll=8)

        copies = []
        for j in range(UPC):
            s_j = (u0 + j) // BLK
            copies.append(pltpu.async_copy(
                pos_hbm.at[s_j], pose_v.at[j], sem))
        for j in range(UPC):
            copies.append(pltpu.async_copy(
                table_hbm.at[gidx_v.at[pl.ds(j * 128, 128)]],
                gbuf.at[pl.ds(j * 128, 128)], sem))
        for cp in copies:
            cp.wait()

        out_fires = []
        for j in range(UPC):
            for b16 in range(8):
                base = j * 128 + b16 * 16
                offv = (idx_v[pl.ds(base, 16)] & 3) * EMB
                rowv = lax.iota(jnp.int32, 16) + base

                def ebody(e, carry2, offv=offv, rowv=rowv, j=j, b16=b16):
                    colv = offv + e
                    vec = plsc.load_gather(gbuf, [rowv, colv])
                    pvec = pose_v[j, pl.ds(e * 16, 16)]
                    obuf[j, e, pl.ds(b16 * 16, 16)] = vec * SCALE + pvec
                    return carry2

                lax.fori_loop(0, EMB, ebody, 0, unroll=4)

        for j in range(UPC):
            s_j = (u0 + j) // BLK
            blk_j = lax.rem(u0 + j, BLK)
            out_fires.append(pltpu.async_copy(
                obuf.at[j],
                out_hbm.at[s_j, :, pl.ds(pl.multiple_of(blk_j * 128, 128), 128)],
                sem_out))
        for cp in out_fires:
            cp.wait()
        return carry

    lax.fori_loop(0, NCHUNK, chunk_body, 0)


def kernel(inputs, token_table, pos_table):
    tail4 = token_table[NCOLS_FULL * 128:].reshape(TAILR, 128)
    table4 = _sc_format(token_table.T, tail4)
    idxT = inputs.T.reshape(-1).astype(jnp.int32)
    pos_exp = jnp.broadcast_to(
        pos_table[:, :, None], (SEQ, EMB, 16)).reshape(SEQ, EMB * 16)
    out = _sc_gather(idxT, table4, pos_exp)
    return out.transpose(2, 0, 1)
